# baseline (speedup 1.0000x reference)
import jax
import jax.numpy as jnp
from jax import lax
from jax.experimental import pallas as pl
from jax.experimental.pallas import tpu as pltpu
from jax.experimental.pallas import tpu_sc as plsc

N_NODES = 100000
N_EDGES = 1600000
HIDDEN = 32
NODE_IN = 128

NC = 2
NS = 16
HALF = HIDDEN // NC
GROUP = 128
NGROUPS = N_EDGES // GROUP
NGPAD = 12504
GPT = NGROUPS // NS
CH = 3
E_CH = CH * GROUP
NBUF = 2
NCH_MAIN = 260
MAIN_GROUPS = NCH_MAIN * CH
N_STATIC = (GPT - MAIN_GROUPS) // CH
ROWS_PER_TILE = N_NODES // NS
ZCHUNK = 125
AGG_PAD = 128


def _sc_scatter_body(col2d_hbm, eat_hbm, agg_hbm, agg_sh,
                     sem_in0, sem_in1, sem_sc0, sem_sc1):
    pl.run_scoped(
        lambda idx_v, in_v, rows_v, zbuf: _sc_scatter_inner(
            col2d_hbm, eat_hbm, agg_hbm, idx_v, in_v, rows_v, zbuf, agg_sh,
            (sem_in0, sem_in1), (sem_sc0, sem_sc1)),
        pltpu.VMEM((NBUF, CH, GROUP), jnp.int32),
        pltpu.VMEM((NBUF, HALF, E_CH), jnp.float32),
        pltpu.VMEM((NBUF, E_CH, HALF), jnp.float32),
        pltpu.VMEM((ZCHUNK, HALF), jnp.float32),
    )


def _sc_scatter_inner(col2d_hbm, eat_hbm, agg_hbm, idx_v, in_v, rows_v, zbuf,
                      agg_sh, sems_in, sems_sc):
    c = lax.axis_index("c")
    s = lax.axis_index("s")
    base = s * GPT + jnp.minimum(s, NGROUPS - GPT * NS)
    count = GPT + (s < NGROUPS - GPT * NS).astype(jnp.int32)

    def _zrow(i, _):
        zbuf[i] = jnp.zeros((HALF,), jnp.float32)
        return 0
    lax.fori_loop(0, ZCHUNK, _zrow, 0)

    def _zcopy(k, _):
        pltpu.sync_copy(zbuf,
                        agg_sh.at[pl.ds(s * ROWS_PER_TILE + k * ZCHUNK, ZCHUNK)])
        return 0
    lax.fori_loop(0, ROWS_PER_TILE // ZCHUNK, _zcopy, 0)
    plsc.subcore_barrier()

    def issue_inputs(q, b):
        g0 = base + q * CH
        pltpu.async_copy(col2d_hbm.at[pl.ds(g0, CH)], idx_v.at[b], sems_in[b])
        pltpu.async_copy(eat_hbm.at[pl.ds(c * HALF, HALF),
                                    pl.ds(g0 * GROUP, E_CH)],
                         in_v.at[b], sems_in[b])

    def wait_inputs(b):
        pltpu.make_async_copy(col2d_hbm.at[pl.ds(0, CH)], idx_v.at[b],
                              sems_in[b]).wait()
        pltpu.make_async_copy(eat_hbm.at[pl.ds(0, HALF), pl.ds(0, E_CH)],
                              in_v.at[b], sems_in[b]).wait()

    def transpose(b, n16):
        def _tp(e16, _):
            ebase = e16 * 16
            eidx = lax.broadcasted_iota(jnp.int32, (16,), 0) + ebase
            for f in range(HALF):
                vals = in_v[b, f, pl.ds(ebase, 16)]
                plsc.store_scatter(rows_v.at[b],
                                   [eidx, jnp.full((16,), f, jnp.int32)], vals)
            return 0
        lax.fori_loop(0, n16, _tp, 0)

    def issue_scatters(b, ngr):
        for j in range(ngr):
            pltpu.async_copy(rows_v.at[b, pl.ds(j * GROUP, GROUP)],
                             agg_sh.at[idx_v.at[b, j]], sems_sc[b], add=True)

    def wait_scatters(b, ngr):
        pltpu.make_async_copy(rows_v.at[b, pl.ds(0, ngr * GROUP)],
                              agg_sh.at[pl.ds(0, ngr * GROUP)],
                              sems_sc[b]).wait()

    def chunk(q, b):
        bn = (b + 1) % NBUF
        wait_inputs(b)

        @pl.when(q >= NBUF - 1)
        def _():
            wait_scatters(bn, CH)

        @pl.when(q + 1 < NCH_MAIN)
        def _():
            issue_inputs(q + 1, bn)

        transpose(b, E_CH // 16)
        issue_scatters(b, CH)

    issue_inputs(0, 0)

    def _pair(p, _):
        chunk(NBUF * p, 0)
        chunk(NBUF * p + 1, 1)
        return 0
    lax.fori_loop(0, NCH_MAIN // NBUF, _pair, 0)
    wait_scatters(1, CH)

    for st in range(N_STATIC):
        issue_inputs(NCH_MAIN + st, 0)
        wait_inputs(0)
        transpose(0, E_CH // 16)
        issue_scatters(0, CH)
        wait_scatters(0, CH)

    def _single(t, _):
        pltpu.sync_copy(col2d_hbm.at[pl.ds(base + t, 1)], idx_v.at[0, pl.ds(0, 1)])
        pltpu.sync_copy(eat_hbm.at[pl.ds(c * HALF, HALF),
                                   pl.ds((base + t) * GROUP, GROUP)],
                        in_v.at[0, :, pl.ds(0, GROUP)])
        transpose(0, GROUP // 16)
        pltpu.sync_copy(rows_v.at[0, pl.ds(0, GROUP)],
                        agg_sh.at[idx_v.at[0, 0]], add=True)
        return 0
    lax.fori_loop(MAIN_GROUPS + N_STATIC * CH, count, _single, 0)

    plsc.subcore_barrier()

    def _wb(k, _):
        r0 = s * ROWS_PER_TILE + k * ZCHUNK
        pltpu.sync_copy(agg_sh.at[pl.ds(r0, ZCHUNK)],
                        agg_hbm.at[pl.ds(r0, ZCHUNK), pl.ds(c * HALF, HALF)])
        return 0
    lax.fori_loop(0, ROWS_PER_TILE // ZCHUNK, _wb, 0)


def _sc_scatter(col2d, eat):
    mesh = plsc.VectorSubcoreMesh(core_axis_name="c", subcore_axis_name="s")
    return pl.kernel(
        _sc_scatter_body,
        out_type=jax.ShapeDtypeStruct((N_NODES, AGG_PAD), jnp.float32),
        mesh=mesh,
        scratch_types=[
            pltpu.VMEM_SHARED((N_NODES + 8, HALF), jnp.float32),
            pltpu.SemaphoreType.DMA,
            pltpu.SemaphoreType.DMA,
            pltpu.SemaphoreType.DMA,
            pltpu.SemaphoreType.DMA,
        ],
        compiler_params=pltpu.CompilerParams(use_tc_tiling_on_sc=False,
                                             needs_layout_passes=False),
    )(col2d, eat)


ROW_BLOCK = 4000


def _mlp_body(x_ref, agg_ref, w1x_ref, w1a_ref, b1_ref, w2_ref, b2_ref, out_ref):
    h = jnp.dot(x_ref[...], w1x_ref[...], preferred_element_type=jnp.float32)
    h = h + jnp.dot(agg_ref[:, :HIDDEN], w1a_ref[...],
                    preferred_element_type=jnp.float32)
    h = jnp.maximum(h + b1_ref[...], 0.0)
    h = jnp.dot(h, w2_ref[...], preferred_element_type=jnp.float32)
    out_ref[...] = jnp.maximum(h + b2_ref[...], 0.0)


def _mlp(x, agg_pad, w1x, w1a, b1, w2, b2):
    n_blocks = N_NODES // ROW_BLOCK
    return pl.pallas_call(
        _mlp_body,
        out_shape=jax.ShapeDtypeStruct((N_NODES, HIDDEN), jnp.float32),
        grid=(n_blocks,),
        in_specs=[
            pl.BlockSpec((ROW_BLOCK, NODE_IN), lambda i: (i, 0)),
            pl.BlockSpec((ROW_BLOCK, AGG_PAD), lambda i: (i, 0)),
            pl.BlockSpec((NODE_IN, HIDDEN), lambda i: (0, 0)),
            pl.BlockSpec((HIDDEN, HIDDEN), lambda i: (0, 0)),
            pl.BlockSpec((1, HIDDEN), lambda i: (0, 0)),
            pl.BlockSpec((HIDDEN, HIDDEN), lambda i: (0, 0)),
            pl.BlockSpec((1, HIDDEN), lambda i: (0, 0)),
        ],
        out_specs=pl.BlockSpec((ROW_BLOCK, HIDDEN), lambda i: (i, 0)),
    )(x, agg_pad, w1x, w1a, b1, w2, b2)


def kernel(x, edge_index, edge_attr, u, batch, W1, b1, W2, b2):
    col2d = edge_index[1].astype(jnp.int32).reshape(NGROUPS, GROUP)
    col2d = jnp.pad(col2d, ((0, NGPAD - NGROUPS), (0, 0)))
    eat = edge_attr.T
    agg_pad = _sc_scatter(col2d, eat)
    return _mlp(x, agg_pad, W1[:NODE_IN], W1[NODE_IN:], b1.reshape(1, HIDDEN),
                W2, b2.reshape(1, HIDDEN))
